# baseline (speedup 1.0000x reference)
import jax
import jax.numpy as jnp
from jax.experimental import pallas as pl
from jax.experimental.pallas import tpu as pltpu

N_TOK = 16384
K_CODES = 8192
D = 64
BLK = 512
NBLK = N_TOK // BLK


def _vq_block(e_ref, w_ref, qst_ref, idx_ref, loss_ref, usage_ref,
              counts_ref, loss_acc_ref):
    b = pl.program_id(0)
    e = e_ref[...]
    w = w_ref[...]

    e2 = jnp.sum(e * e, axis=1, keepdims=True)
    m = jax.lax.dot_general(e.astype(jnp.bfloat16), w.astype(jnp.bfloat16),
                            (((1,), (1,)), ((), ())),
                            preferred_element_type=jnp.float32)
    w2 = jnp.sum(w * w, axis=1)
    d = (e2 - 2.0 * m) + w2

    col = jax.lax.broadcasted_iota(jnp.int32, d.shape, 1)
    idx = jnp.argmin(d, axis=1).astype(jnp.int32)
    idx_ref[...] = idx.reshape(1, 1, BLK)

    hit = col == idx[:, None]
    q = jax.lax.dot_general(hit.astype(jnp.bfloat16), w.astype(jnp.bfloat16),
                            (((1,), (0,)), ((), ())),
                            preferred_element_type=jnp.float32)
    qst_ref[...] = e + (q - e)

    blk_counts = jnp.sum(hit.astype(jnp.float32), axis=0).reshape(1, K_CODES)
    blk_loss = jnp.sum((q - e) ** 2).reshape(1, 1)

    @pl.when(b == 0)
    def _():
        counts_ref[...] = blk_counts
        loss_acc_ref[...] = blk_loss

    @pl.when(b > 0)
    def _():
        counts_ref[...] += blk_counts
        loss_acc_ref[...] += blk_loss

    @pl.when(b == NBLK - 1)
    def _():
        counts = counts_ref[...]
        loss_ref[...] = loss_acc_ref[...] / (N_TOK * D)
        usage_ref[...] = (jnp.max(counts) / jnp.sum(counts)).reshape(1, 1)


def _vq(flat, W, interpret=False):
    return pl.pallas_call(
        _vq_block,
        grid=(NBLK,),
        in_specs=[
            pl.BlockSpec((BLK, D), lambda b: (b, 0)),
            pl.BlockSpec((K_CODES, D), lambda b: (0, 0)),
        ],
        out_specs=[
            pl.BlockSpec((BLK, D), lambda b: (b, 0)),
            pl.BlockSpec((1, 1, BLK), lambda b: (b, 0, 0)),
            pl.BlockSpec((1, 1), lambda b: (0, 0)),
            pl.BlockSpec((1, 1), lambda b: (0, 0)),
        ],
        out_shape=[
            jax.ShapeDtypeStruct((N_TOK, D), jnp.float32),
            jax.ShapeDtypeStruct((NBLK, 1, BLK), jnp.int32),
            jax.ShapeDtypeStruct((1, 1), jnp.float32),
            jax.ShapeDtypeStruct((1, 1), jnp.float32),
        ],
        scratch_shapes=[
            pltpu.VMEM((1, K_CODES), jnp.float32),
            pltpu.VMEM((1, 1), jnp.float32),
        ],
        interpret=interpret,
    )(flat, W)


def kernel(e, W):
    flat = e.reshape(-1, D)
    qst, idx3, loss, usage = _vq(flat, W)
    return (qst.reshape(e.shape), idx3.reshape(N_TOK),
            loss[0, 0], usage[0, 0])
